# baseline (speedup 1.0000x reference)
import functools

import jax
import jax.numpy as jnp
from jax import lax
from jax.experimental import pallas as pl
from jax.experimental.pallas import tpu as pltpu
from jax.experimental.pallas import tpu_sc as plsc

N = 10000
E = 320000
D = 128

_TBLK = 1000


def _round_bf16_bits(x):
    b = lax.bitcast_convert_type(x, jnp.uint32)
    return (b + jnp.uint32(0x7FFF) + ((b >> jnp.uint32(16)) & jnp.uint32(1))) \
        >> jnp.uint32(16)


def _table_body(nodes_ref, rel_ref, dis_ref, disrel_ref, out_ref):
    h = jnp.dot(nodes_ref[...], rel_ref[...], preferred_element_type=jnp.float32)
    d = jnp.dot(dis_ref[...], disrel_ref[...], preferred_element_type=jnp.float32)
    packed = (_round_bf16_bits(d) << jnp.uint32(16)) | _round_bf16_bits(h)
    out_ref[...] = lax.bitcast_convert_type(packed, jnp.int32)


def _build_table(nodes_embedding, relation_matrix, dis_node_emb,
                 dis_relation_matrix):
    return pl.pallas_call(
        _table_body,
        grid=(N // _TBLK,),
        in_specs=[
            pl.BlockSpec((_TBLK, D), lambda i: (i, 0)),
            pl.BlockSpec((D, D), lambda i: (0, 0)),
            pl.BlockSpec((_TBLK, D), lambda i: (i, 0)),
            pl.BlockSpec((D, D), lambda i: (0, 0)),
        ],
        out_specs=pl.BlockSpec((_TBLK, D), lambda i: (i, 0)),
        out_shape=jax.ShapeDtypeStruct((N, D), jnp.int32),
    )(nodes_embedding, relation_matrix, dis_node_emb, dis_relation_matrix)


_NC, _NS = 2, 16
_NW = _NC * _NS
_EPW = E // _NW
_CH = 40
_KG = 5
_GR = _KG * _CH
_NGRP = _EPW // _GR
_NB = 4


def _gather_body(table_hbm, idx_hbm, out_hbm, idx_v, rows_v, *sems):
    sem_g, sem_w = sems[:_NB], sems[_NB:]
    wid = lax.axis_index("s") * _NC + lax.axis_index("c")
    wbase = wid * _EPW

    def start_gather(b, g):
        for j in range(_KG):
            c = g * _KG + j
            pltpu.async_copy(
                table_hbm.at[idx_v.at[pl.ds(c * _CH, _CH)]],
                rows_v.at[b, pl.ds(j * _CH, _CH)], sem_g[b])

    def wait_gather(b):
        pltpu.make_async_copy(out_hbm.at[pl.ds(0, _GR)], rows_v.at[b],
                              sem_g[b]).wait()

    def start_write(b, g):
        return pltpu.async_copy(rows_v.at[b],
                                out_hbm.at[pl.ds(wbase + g * _GR, _GR)],
                                sem_w[b])

    def wait_write(b):
        pltpu.make_async_copy(rows_v.at[b], out_hbm.at[pl.ds(wbase, _GR)],
                              sem_w[b]).wait()

    pltpu.sync_copy(idx_hbm.at[pl.ds(wbase, _EPW)], idx_v)
    start_gather(0, 0)
    start_gather(1, 1)

    def quad(i, carry):
        for k in range(_NB):
            g = _NB * i + k
            bn = (k + 2) % _NB
            if k < 2:
                @pl.when(i > 0)
                def _():
                    wait_write(bn)
            else:
                wait_write(bn)
            start_gather(bn, g + 2)
            wait_gather(k)
            start_write(k, g)
        return carry

    lax.fori_loop(0, (_NGRP - 2) // _NB, quad, 0)
    wait_gather(0)
    w0 = start_write(0, _NGRP - 2)
    wait_gather(1)
    w1 = start_write(1, _NGRP - 1)
    wait_write(2)
    wait_write(3)
    w0.wait()
    w1.wait()


def _gather(table, src):
    k = functools.partial(
        pl.kernel,
        mesh=plsc.VectorSubcoreMesh(core_axis_name="c", subcore_axis_name="s"),
        out_type=jax.ShapeDtypeStruct((E, D), jnp.int32),
        scratch_types=[
            pltpu.VMEM((_EPW,), jnp.int32),
            pltpu.VMEM((_NB, _GR, D), jnp.int32),
        ] + [pltpu.SemaphoreType.DMA] * (2 * _NB),
    )(_gather_body)
    return k(table, src)


_EBLK = 16000
_NEB = E // _EBLK


def _lrelu(x):
    return jnp.maximum(x, 0.01 * x)


def _mlp_body(g_ref, noise_ref, w1t_ref, b1_ref, w2t_ref, b2_ref, out_ref):
    gu = lax.bitcast_convert_type(g_ref[...], jnp.uint32)
    gh = lax.bitcast_convert_type(gu << jnp.uint32(16), jnp.float32)
    gd = lax.bitcast_convert_type(gu & jnp.uint32(0xFFFF0000), jnp.float32)
    g0 = gh + noise_ref[...]
    z1 = jnp.dot(g0, w1t_ref[...], preferred_element_type=jnp.float32) + b1_ref[...]
    g1 = _lrelu(z1)
    z2 = jnp.dot(g1, w2t_ref[...], preferred_element_type=jnp.float32) + b2_ref[...]
    g2 = _lrelu(z2)
    prod = gd * g2
    ones = jnp.ones((1, D), jnp.float32)
    score = lax.dot_general(ones, prod, (((1,), (1,)), ((), ())),
                            preferred_element_type=jnp.float32)
    out_ref[...] = score[None, :, :]


def _mlp_score(g, noise_emb, w1t, b1, w2t, b2):
    return pl.pallas_call(
        _mlp_body,
        grid=(_NEB,),
        in_specs=[
            pl.BlockSpec((_EBLK, D), lambda i: (i, 0)),
            pl.BlockSpec((_EBLK, D), lambda i: (i, 0)),
            pl.BlockSpec((D, D), lambda i: (0, 0)),
            pl.BlockSpec((1, D), lambda i: (0, 0)),
            pl.BlockSpec((D, D), lambda i: (0, 0)),
            pl.BlockSpec((1, D), lambda i: (0, 0)),
        ],
        out_specs=pl.BlockSpec((1, 1, _EBLK), lambda i: (i, 0, 0)),
        out_shape=jax.ShapeDtypeStruct((_NEB, 1, _EBLK), jnp.float32),
    )(g, noise_emb, w1t, b1, w2t, b2)


def kernel(edge_index, dis_node_emb, dis_relation_matrix, noise_emb,
           nodes_embedding, relation_matrix, W1, b1, W2, b2):
    src = edge_index[0]
    table = _build_table(nodes_embedding, relation_matrix, dis_node_emb,
                         dis_relation_matrix)
    g = _gather(table, src)
    score = _mlp_score(g, noise_emb, W1.T, b1.reshape(1, D), W2.T,
                       b2.reshape(1, D))
    return score.reshape(E)


# speedup vs baseline: 1.0027x; 1.0027x over previous
import functools

import jax
import jax.numpy as jnp
from jax import lax
from jax.experimental import pallas as pl
from jax.experimental.pallas import tpu as pltpu
from jax.experimental.pallas import tpu_sc as plsc

N = 10000
E = 320000
D = 128

_TBLK = 1000


def _round_bf16_bits(x):
    b = lax.bitcast_convert_type(x, jnp.uint32)
    return (b + jnp.uint32(0x7FFF) + ((b >> jnp.uint32(16)) & jnp.uint32(1))) \
        >> jnp.uint32(16)


def _table_body(nodes_ref, rel_ref, dis_ref, disrel_ref, out_ref):
    h = jnp.dot(nodes_ref[...], rel_ref[...], preferred_element_type=jnp.float32)
    d = jnp.dot(dis_ref[...], disrel_ref[...], preferred_element_type=jnp.float32)
    packed = (_round_bf16_bits(d) << jnp.uint32(16)) | _round_bf16_bits(h)
    out_ref[...] = lax.bitcast_convert_type(packed, jnp.int32)


def _build_table(nodes_embedding, relation_matrix, dis_node_emb,
                 dis_relation_matrix):
    return pl.pallas_call(
        _table_body,
        grid=(N // _TBLK,),
        in_specs=[
            pl.BlockSpec((_TBLK, D), lambda i: (i, 0)),
            pl.BlockSpec((D, D), lambda i: (0, 0)),
            pl.BlockSpec((_TBLK, D), lambda i: (i, 0)),
            pl.BlockSpec((D, D), lambda i: (0, 0)),
        ],
        out_specs=pl.BlockSpec((_TBLK, D), lambda i: (i, 0)),
        out_shape=jax.ShapeDtypeStruct((N, D), jnp.int32),
    )(nodes_embedding, relation_matrix, dis_node_emb, dis_relation_matrix)


_NC, _NS = 2, 16
_NW = _NC * _NS
_EPW = E // _NW
_CH = 80
_KG = 5
_GR = _KG * _CH
_NGRP = _EPW // _GR


def _gather_body(table_hbm, idx_hbm, out_hbm, idx_v, rows_v, sem_g0, sem_g1,
                 sem_w0, sem_w1):
    wid = lax.axis_index("s") * _NC + lax.axis_index("c")
    wbase = wid * _EPW

    def start_gather(b, g, sem):
        for j in range(_KG):
            c = g * _KG + j
            pltpu.async_copy(
                table_hbm.at[idx_v.at[pl.ds(c * _CH, _CH)]],
                rows_v.at[b, pl.ds(j * _CH, _CH)], sem)

    def wait_gather(b, sem):
        pltpu.make_async_copy(out_hbm.at[pl.ds(0, _GR)], rows_v.at[b],
                              sem).wait()

    def start_write(b, g, sem):
        return pltpu.async_copy(rows_v.at[b],
                                out_hbm.at[pl.ds(wbase + g * _GR, _GR)], sem)

    def wait_write(b, sem):
        pltpu.make_async_copy(rows_v.at[b], out_hbm.at[pl.ds(wbase, _GR)],
                              sem).wait()

    pltpu.sync_copy(idx_hbm.at[pl.ds(wbase, _EPW)], idx_v)
    start_gather(0, 0, sem_g0)

    def pair(i, carry):
        g0 = 2 * i
        @pl.when(i > 0)
        def _():
            wait_write(1, sem_w1)
        start_gather(1, g0 + 1, sem_g1)
        wait_gather(0, sem_g0)
        start_write(0, g0, sem_w0).wait()
        start_gather(0, g0 + 2, sem_g0)
        wait_gather(1, sem_g1)
        start_write(1, g0 + 1, sem_w1)
        return carry

    lax.fori_loop(0, _NGRP // 2, pair, 0)
    wait_write(1, sem_w1)
    wait_gather(0, sem_g0)
    start_write(0, _NGRP - 1, sem_w0).wait()


def _gather(table, src):
    k = functools.partial(
        pl.kernel,
        mesh=plsc.VectorSubcoreMesh(core_axis_name="c", subcore_axis_name="s"),
        out_type=jax.ShapeDtypeStruct((E, D), jnp.int32),
        scratch_types=[
            pltpu.VMEM((_EPW,), jnp.int32),
            pltpu.VMEM((2, _GR, D), jnp.int32),
            pltpu.SemaphoreType.DMA,
            pltpu.SemaphoreType.DMA,
            pltpu.SemaphoreType.DMA,
            pltpu.SemaphoreType.DMA,
        ],
    )(_gather_body)
    return k(table, src)


_EBLK = 16000
_NEB = E // _EBLK


def _lrelu(x):
    return jnp.maximum(x, 0.01 * x)


def _mlp_body(g_ref, noise_ref, w1t_ref, b1_ref, w2t_ref, b2_ref, out_ref):
    gu = lax.bitcast_convert_type(g_ref[...], jnp.uint32)
    gh = lax.bitcast_convert_type(gu << jnp.uint32(16), jnp.float32)
    gd = lax.bitcast_convert_type(gu & jnp.uint32(0xFFFF0000), jnp.float32)
    g0 = gh + noise_ref[...]
    z1 = jnp.dot(g0, w1t_ref[...], preferred_element_type=jnp.float32) + b1_ref[...]
    g1 = _lrelu(z1)
    z2 = jnp.dot(g1, w2t_ref[...], preferred_element_type=jnp.float32) + b2_ref[...]
    g2 = _lrelu(z2)
    prod = gd * g2
    ones = jnp.ones((1, D), jnp.float32)
    score = lax.dot_general(ones, prod, (((1,), (1,)), ((), ())),
                            preferred_element_type=jnp.float32)
    out_ref[...] = score[None, :, :]


def _mlp_score(g, noise_emb, w1t, b1, w2t, b2):
    return pl.pallas_call(
        _mlp_body,
        grid=(_NEB,),
        in_specs=[
            pl.BlockSpec((_EBLK, D), lambda i: (i, 0)),
            pl.BlockSpec((_EBLK, D), lambda i: (i, 0)),
            pl.BlockSpec((D, D), lambda i: (0, 0)),
            pl.BlockSpec((1, D), lambda i: (0, 0)),
            pl.BlockSpec((D, D), lambda i: (0, 0)),
            pl.BlockSpec((1, D), lambda i: (0, 0)),
        ],
        out_specs=pl.BlockSpec((1, 1, _EBLK), lambda i: (i, 0, 0)),
        out_shape=jax.ShapeDtypeStruct((_NEB, 1, _EBLK), jnp.float32),
    )(g, noise_emb, w1t, b1, w2t, b2)


def kernel(edge_index, dis_node_emb, dis_relation_matrix, noise_emb,
           nodes_embedding, relation_matrix, W1, b1, W2, b2):
    src = edge_index[0]
    table = _build_table(nodes_embedding, relation_matrix, dis_node_emb,
                         dis_relation_matrix)
    g = _gather(table, src)
    score = _mlp_score(g, noise_emb, W1.T, b1.reshape(1, D), W2.T,
                       b2.reshape(1, D))
    return score.reshape(E)
